# Initial kernel scaffold; baseline (speedup 1.0000x reference)
import jax
import jax.numpy as jnp
from jax.experimental import pallas as pl


def kernel(x, edge_index, W1, b1, W2, b2, Wmu, bmu, Wlv, blv):
    raise NotImplementedError("write your pallas kernel here")


# speedup vs baseline: 14.4629x; 14.4629x over previous
import functools

import jax
import jax.numpy as jnp
from jax import lax
from jax.experimental import pallas as pl
from jax.experimental.pallas import tpu as pltpu
from jax.experimental.pallas import tpu_sc as plsc

NC, NS, LANES = 2, 16, 16
NW = NC * NS
CH = 128

_mesh = lambda: plsc.VectorSubcoreMesh(core_axis_name="c", subcore_axis_name="s",
                                       num_cores=NC, num_subcores=NS)


def _deg_call(dst_pad, zeros_1d, e_pad, n_acc):
    per_w = e_pad // NW
    nchunk = per_w // CH

    def body(dst_hbm, z_hbm, out_hbm, ones_v, idx_d, acc_sh, sem):
        cid = lax.axis_index("c")
        sid = lax.axis_index("s")
        wid = cid * NS + sid

        @pl.when(sid == 0)
        def _():
            pltpu.sync_copy(z_hbm, acc_sh)

        for j in range(CH // LANES):
            ones_v[pl.ds(j * LANES, LANES)] = jnp.ones((LANES,), jnp.float32)
        plsc.subcore_barrier()

        def chunk(i, carry):
            base = pl.multiple_of(wid * per_w + i * CH, CH)
            pltpu.sync_copy(dst_hbm.at[pl.ds(base, CH)], idx_d)
            pltpu.sync_copy(ones_v, acc_sh.at[idx_d], add=True)
            return carry

        lax.fori_loop(0, nchunk, chunk, 0)
        plsc.subcore_barrier()

        @pl.when(sid == 0)
        def _():
            pltpu.sync_copy(acc_sh, out_hbm.at[pl.ds(cid * n_acc, n_acc)])

    f = pl.kernel(
        body,
        out_type=jax.ShapeDtypeStruct((NC * n_acc,), jnp.float32),
        mesh=_mesh(),
        scratch_types=[
            pltpu.VMEM((CH,), jnp.float32),
            pltpu.VMEM((CH,), jnp.int32),
            pltpu.VMEM_SHARED((n_acc,), jnp.float32),
            pltpu.SemaphoreType.DMA,
        ],
    )
    return f(dst_pad, zeros_1d)


def _conv_call(g, zeros_2d, src_pad, dst_pad, e_pad, n_acc):
    n, d = g.shape
    per_w = e_pad // NW
    nchunk = per_w // CH

    def body(g_hbm, z_hbm, src_hbm, dst_hbm, out_hbm,
             idx_s, idx_d, rows, acc_sh, sem):
        cid = lax.axis_index("c")
        sid = lax.axis_index("s")
        wid = cid * NS + sid

        @pl.when((sid == 0) & (cid == 0))
        def _():
            pltpu.sync_copy(g_hbm, acc_sh.at[pl.ds(0, n)])

        @pl.when((sid == 0) & (cid != 0))
        def _():
            pltpu.sync_copy(z_hbm, acc_sh.at[pl.ds(0, n)])

        plsc.subcore_barrier()

        def chunk(i, carry):
            base = pl.multiple_of(wid * per_w + i * CH, CH)
            pltpu.sync_copy(src_hbm.at[pl.ds(base, CH)], idx_s)
            pltpu.sync_copy(dst_hbm.at[pl.ds(base, CH)], idx_d)
            pltpu.async_copy(g_hbm.at[idx_s], rows, sem).wait()
            pltpu.sync_copy(rows, acc_sh.at[idx_d], add=True)
            return carry

        lax.fori_loop(0, nchunk, chunk, 0)
        plsc.subcore_barrier()

        @pl.when(sid == 0)
        def _():
            pltpu.sync_copy(acc_sh.at[pl.ds(0, n)], out_hbm.at[cid])

    f = pl.kernel(
        body,
        out_type=jax.ShapeDtypeStruct((NC, n, d), jnp.float32),
        mesh=_mesh(),
        compiler_params=pltpu.CompilerParams(use_tc_tiling_on_sc=False),
        scratch_types=[
            pltpu.VMEM((CH,), jnp.int32),
            pltpu.VMEM((CH,), jnp.int32),
            pltpu.VMEM((CH, d), jnp.float32),
            pltpu.VMEM_SHARED((n_acc, d), jnp.float32),
            pltpu.SemaphoreType.DMA,
        ],
    )
    return f(g, zeros_2d, src_pad, dst_pad)


_BR = 1000


def _tc_first(deg_t, x, w1):
    n, in_dim = x.shape
    d = w1.shape[1]
    grid = (n // _BR,)

    def body(dp_ref, x_ref, w_ref, dis_ref, g_ref):
        dp = dp_ref[...]
        dis = lax.rsqrt(dp[:, :1] + dp[:, 1:] + 1.0)
        dis_ref[...] = dis
        g_ref[...] = dis * jnp.dot(x_ref[...], w_ref[...],
                                   preferred_element_type=jnp.float32)

    return pl.pallas_call(
        body,
        grid=grid,
        in_specs=[
            pl.BlockSpec((_BR, NC), lambda i: (i, 0)),
            pl.BlockSpec((_BR, in_dim), lambda i: (i, 0)),
            pl.BlockSpec((in_dim, d), lambda i: (0, 0)),
        ],
        out_specs=[
            pl.BlockSpec((_BR, 1), lambda i: (i, 0)),
            pl.BlockSpec((_BR, d), lambda i: (i, 0)),
        ],
        out_shape=[
            jax.ShapeDtypeStruct((n, 1), jnp.float32),
            jax.ShapeDtypeStruct((n, d), jnp.float32),
        ],
    )(deg_t, x, w1)


def _tc_mid(p, dis, b, w):
    _, n, d = p.shape
    d2 = w.shape[1]

    def body(p_ref, dis_ref, b_ref, w_ref, g_ref):
        dv = dis_ref[...]
        h = jnp.maximum(dv * (p_ref[0] + p_ref[1]) + b_ref[...], 0.0)
        g_ref[...] = dv * jnp.dot(h, w_ref[...],
                                  preferred_element_type=jnp.float32)

    return pl.pallas_call(
        body,
        grid=(n // _BR,),
        in_specs=[
            pl.BlockSpec((NC, _BR, d), lambda i: (0, i, 0)),
            pl.BlockSpec((_BR, 1), lambda i: (i, 0)),
            pl.BlockSpec((1, d), lambda i: (0, 0)),
            pl.BlockSpec((d, d2), lambda i: (0, 0)),
        ],
        out_specs=pl.BlockSpec((_BR, d2), lambda i: (i, 0)),
        out_shape=jax.ShapeDtypeStruct((n, d2), jnp.float32),
    )(p, dis, b, w)


def _tc_final(p, dis, b):
    _, n, d = p.shape

    def body(p_ref, dis_ref, b_ref, o_ref):
        o_ref[...] = dis_ref[...] * (p_ref[0] + p_ref[1]) + b_ref[...]

    return pl.pallas_call(
        body,
        grid=(n // _BR,),
        in_specs=[
            pl.BlockSpec((NC, _BR, d), lambda i: (0, i, 0)),
            pl.BlockSpec((_BR, 1), lambda i: (i, 0)),
            pl.BlockSpec((1, d), lambda i: (0, 0)),
        ],
        out_specs=pl.BlockSpec((_BR, d), lambda i: (i, 0)),
        out_shape=jax.ShapeDtypeStruct((n, d), jnp.float32),
    )(p, dis, b)


def kernel(x, edge_index, W1, b1, W2, b2, Wmu, bmu, Wlv, blv):
    n = x.shape[0]
    e = edge_index.shape[1]
    d = W1.shape[1]

    e_pad = -(-e // (NW * CH)) * (NW * CH)
    pad = e_pad - e
    n_acc = n + LANES
    n_acc1 = -(-(n + 1) // 128) * 128

    src = edge_index[0].astype(jnp.int32)
    dst = edge_index[1].astype(jnp.int32)
    src_pad = jnp.concatenate([src, jnp.zeros((pad,), jnp.int32)])
    dst_pad = jnp.concatenate([dst, jnp.full((pad,), n, jnp.int32)])

    zeros_1d = jnp.zeros((n_acc1,), jnp.float32)
    zeros_2d = jnp.zeros((n, d), jnp.float32)

    deg_p = _deg_call(dst_pad, zeros_1d, e_pad, n_acc1)
    dis, g1 = _tc_first(deg_p.reshape(NC, n_acc1)[:, :n].T, x, W1)

    p1 = _conv_call(g1, zeros_2d, src_pad, dst_pad, e_pad, n_acc)
    g2 = _tc_mid(p1, dis, b1.reshape(1, d), W2)

    p2 = _conv_call(g2, zeros_2d, src_pad, dst_pad, e_pad, n_acc)
    wc = jnp.concatenate([Wmu, Wlv], axis=1)
    g3 = _tc_mid(p2, dis, b2.reshape(1, d), wc)

    p3 = _conv_call(g3, zeros_2d, src_pad, dst_pad, e_pad, n_acc)
    bc = jnp.concatenate([bmu, blv]).reshape(1, d)
    out = _tc_final(p3, dis, bc)

    z = Wmu.shape[1]
    return (out[:, :z], out[:, z:])
